# Initial kernel scaffold; baseline (speedup 1.0000x reference)
import jax
import jax.numpy as jnp
from jax.experimental import pallas as pl


def kernel(x, t, embeddings):
    raise NotImplementedError("write your pallas kernel here")


# speedup vs baseline: 2.3455x; 2.3455x over previous
import functools

import jax
import jax.numpy as jnp
from jax import lax
from jax.experimental import pallas as pl
from jax.experimental.pallas import tpu as pltpu
from jax.experimental.pallas import tpu_sc as plsc

_TIME_STEPS = 1000
_EMBED_DIM = 128
_BATCH = 16384

_NUM_CORES = 2
_NUM_SUBCORES = 16
_NUM_WORKERS = _NUM_CORES * _NUM_SUBCORES
_B_PER_W = _BATCH // _NUM_WORKERS
_CHUNK = 128
_NCHUNKS = _B_PER_W // _CHUNK


def _gather_kernel(table_hbm, idx_hbm, out_hbm, idx_v, rows_v, sem):
    wid = lax.axis_index("s") * _NUM_CORES + lax.axis_index("c")
    base = wid * _B_PER_W
    for j in range(_NCHUNKS):
        pltpu.sync_copy(idx_hbm.at[pl.ds(base + j * _CHUNK, _CHUNK)],
                        idx_v.at[j])
    copies = []
    for j in range(_NCHUNKS):
        copies.append(
            pltpu.async_copy(table_hbm.at[idx_v.at[j]],
                             rows_v.at[pl.ds(j * _CHUNK, _CHUNK)], sem))
    for c in copies:
        c.wait()
    pltpu.sync_copy(rows_v, out_hbm.at[pl.ds(base, _B_PER_W)])


@jax.jit
def _gather(embeddings, t):
    mesh = plsc.VectorSubcoreMesh(core_axis_name="c", subcore_axis_name="s")
    return pl.kernel(
        _gather_kernel,
        mesh=mesh,
        out_type=jax.ShapeDtypeStruct((_BATCH, _EMBED_DIM), jnp.float32),
        scratch_types=[
            pltpu.VMEM((_NCHUNKS, _CHUNK), jnp.int32),
            pltpu.VMEM((_B_PER_W, _EMBED_DIM), jnp.float32),
            pltpu.SemaphoreType.DMA,
        ],
    )(embeddings, t)


def kernel(x, t, embeddings):
    return _gather(embeddings, t)


# speedup vs baseline: 2.3634x; 1.0076x over previous
import functools

import jax
import jax.numpy as jnp
from jax import lax
from jax.experimental import pallas as pl
from jax.experimental.pallas import tpu as pltpu
from jax.experimental.pallas import tpu_sc as plsc

_TIME_STEPS = 1000
_EMBED_DIM = 128
_BATCH = 16384

_NUM_CORES = 2
_NUM_SUBCORES = 16
_NUM_WORKERS = _NUM_CORES * _NUM_SUBCORES
_B_PER_W = _BATCH // _NUM_WORKERS
_CHUNK = 128
_NCHUNKS = _B_PER_W // _CHUNK


def _gather_kernel(table_hbm, idx_hbm, out_hbm, idx_v, rows_v,
                   sem_g0, sem_g1, sem_g2, sem_g3, sem_w):
    sems = (sem_g0, sem_g1, sem_g2, sem_g3)
    wid = lax.axis_index("s") * _NUM_CORES + lax.axis_index("c")
    base = wid * _B_PER_W
    pltpu.sync_copy(idx_hbm.at[wid], idx_v)
    gathers = []
    for j in range(_NCHUNKS):
        gathers.append(
            pltpu.async_copy(table_hbm.at[idx_v.at[j]],
                             rows_v.at[pl.ds(j * _CHUNK, _CHUNK)], sems[j]))
    writes = []
    for j in range(_NCHUNKS):
        gathers[j].wait()
        writes.append(
            pltpu.async_copy(rows_v.at[pl.ds(j * _CHUNK, _CHUNK)],
                             out_hbm.at[pl.ds(base + j * _CHUNK, _CHUNK)],
                             sem_w))
    for w in writes:
        w.wait()


@jax.jit
def _gather(embeddings, t):
    mesh = plsc.VectorSubcoreMesh(core_axis_name="c", subcore_axis_name="s")
    t3 = t.reshape(_NUM_WORKERS, _NCHUNKS, _CHUNK)
    return pl.kernel(
        _gather_kernel,
        mesh=mesh,
        out_type=jax.ShapeDtypeStruct((_BATCH, _EMBED_DIM), jnp.float32),
        scratch_types=[
            pltpu.VMEM((_NCHUNKS, _CHUNK), jnp.int32),
            pltpu.VMEM((_B_PER_W, _EMBED_DIM), jnp.float32),
            pltpu.SemaphoreType.DMA,
            pltpu.SemaphoreType.DMA,
            pltpu.SemaphoreType.DMA,
            pltpu.SemaphoreType.DMA,
            pltpu.SemaphoreType.DMA,
        ],
    )(embeddings, t3)


def kernel(x, t, embeddings):
    return _gather(embeddings, t)


# speedup vs baseline: 2.4406x; 1.0326x over previous
import functools

import jax
import jax.numpy as jnp
from jax import lax
from jax.experimental import pallas as pl
from jax.experimental.pallas import tpu as pltpu
from jax.experimental.pallas import tpu_sc as plsc

_TIME_STEPS = 1000
_EMBED_DIM = 128
_BATCH = 16384

_NUM_CORES = 2
_NUM_SUBCORES = 16
_NUM_WORKERS = _NUM_CORES * _NUM_SUBCORES
_B_PER_W = _BATCH // _NUM_WORKERS
_CHUNK = 128
_NCHUNKS = _B_PER_W // _CHUNK


def _gather_kernel(table_hbm, idx_hbm, out_hbm, idx_v, rows_v, sem):
    wid = lax.axis_index("s") * _NUM_CORES + lax.axis_index("c")
    base = wid * _B_PER_W
    pltpu.sync_copy(idx_hbm.at[wid], idx_v)
    pltpu.async_copy(table_hbm.at[idx_v], rows_v, sem).wait()
    pltpu.sync_copy(rows_v, out_hbm.at[pl.ds(base, _B_PER_W)])


@jax.jit
def _gather(embeddings, t):
    mesh = plsc.VectorSubcoreMesh(core_axis_name="c", subcore_axis_name="s")
    t2 = t.reshape(_NUM_WORKERS, _B_PER_W)
    return pl.kernel(
        _gather_kernel,
        mesh=mesh,
        out_type=jax.ShapeDtypeStruct((_BATCH, _EMBED_DIM), jnp.float32),
        scratch_types=[
            pltpu.VMEM((_B_PER_W,), jnp.int32),
            pltpu.VMEM((_B_PER_W, _EMBED_DIM), jnp.float32),
            pltpu.SemaphoreType.DMA,
        ],
    )(embeddings, t2)


def kernel(x, t, embeddings):
    return _gather(embeddings, t)


# speedup vs baseline: 2.4669x; 1.0108x over previous
import functools

import jax
import jax.numpy as jnp
from jax import lax
from jax.experimental import pallas as pl
from jax.experimental.pallas import tpu as pltpu
from jax.experimental.pallas import tpu_sc as plsc

_TIME_STEPS = 1000
_EMBED_DIM = 128
_BATCH = 16384

_NUM_CORES = 2
_NUM_SUBCORES = 16
_NUM_WORKERS = _NUM_CORES * _NUM_SUBCORES
_B_PER_W = _BATCH // _NUM_WORKERS
_CHUNK = 128
_NCHUNKS = _B_PER_W // _CHUNK


def _gather_kernel(table_hbm, idx_hbm, out_hbm, idx_v, rows_v, sem):
    wid = lax.axis_index("s") * _NUM_CORES + lax.axis_index("c")
    base = wid * _B_PER_W
    pltpu.sync_copy(idx_hbm.at[pl.ds(base, _B_PER_W)], idx_v)
    pltpu.async_copy(table_hbm.at[idx_v], rows_v, sem).wait()
    pltpu.sync_copy(rows_v, out_hbm.at[pl.ds(base, _B_PER_W)])


@jax.jit
def _gather(embeddings, t):
    mesh = plsc.VectorSubcoreMesh(core_axis_name="c", subcore_axis_name="s")
    return pl.kernel(
        _gather_kernel,
        mesh=mesh,
        out_type=jax.ShapeDtypeStruct((_BATCH, _EMBED_DIM), jnp.float32),
        scratch_types=[
            pltpu.VMEM((_B_PER_W,), jnp.int32),
            pltpu.VMEM((_B_PER_W, _EMBED_DIM), jnp.float32),
            pltpu.SemaphoreType.DMA,
        ],
    )(embeddings, t)


def kernel(x, t, embeddings):
    return _gather(embeddings, t)
